# Initial kernel scaffold; baseline (speedup 1.0000x reference)
import jax
import jax.numpy as jnp
from jax.experimental import pallas as pl


def kernel(x, W):
    raise NotImplementedError("write your pallas kernel here")


# speedup vs baseline: 2.0510x; 2.0510x over previous
import jax
import jax.numpy as jnp
from jax.experimental import pallas as pl
from jax.experimental.pallas import tpu as pltpu

K = 512
D = 256
BETA = 0.25

_TILE = 1536


def _vq_body(x_ref, w_ref, enc_ref, q_ref, loss_ref):
    i = pl.program_id(0)
    xt = x_ref[...]
    w = w_ref[...]
    x_sq = jnp.sum(xt * xt, axis=1, keepdims=True)
    w_sq = jnp.sum(w * w, axis=1)
    mm = jax.lax.dot_general(xt, w, (((1,), (1,)), ((), ())),
                             preferred_element_type=jnp.float32)
    d = x_sq + w_sq - 2.0 * mm
    dmin = jnp.min(d, axis=1, keepdims=True)
    iota_k = jax.lax.broadcasted_iota(jnp.int32, d.shape, 1)
    idx = jnp.min(jnp.where(d == dmin, iota_k, K), axis=1)
    enc = (idx[:, None] == iota_k).astype(jnp.float32)
    enc_ref[...] = enc
    q_ref[...] = jax.lax.dot_general(enc, w, (((1,), (0,)), ((), ())),
                                     preferred_element_type=jnp.float32)

    @pl.when(i == 0)
    def _():
        loss_ref[...] = jnp.zeros((1, 1), jnp.float32)

    loss_ref[...] += jnp.sum(dmin)[None, None]


def kernel(x, W):
    B, C, D1, D2, D3 = x.shape
    N = B * D1 * D2 * D3
    x_flat = jnp.transpose(x, (0, 2, 3, 4, 1)).reshape(N, D)
    grid = (N // _TILE,)
    enc, quant, loss_sum = pl.pallas_call(
        _vq_body,
        grid=grid,
        in_specs=[
            pl.BlockSpec((_TILE, D), lambda i: (i, 0)),
            pl.BlockSpec((K, D), lambda i: (0, 0)),
        ],
        out_specs=[
            pl.BlockSpec((_TILE, K), lambda i: (i, 0)),
            pl.BlockSpec((_TILE, D), lambda i: (i, 0)),
            pl.BlockSpec((1, 1), lambda i: (0, 0)),
        ],
        out_shape=[
            jax.ShapeDtypeStruct((N, K), jnp.float32),
            jax.ShapeDtypeStruct((N, D), jnp.float32),
            jax.ShapeDtypeStruct((1, 1), jnp.float32),
        ],
    )(x_flat, W)
    mse = loss_sum[0, 0] / (N * D)
    e_latent = jnp.clip(mse, 0.0, 10.0)
    loss = e_latent + BETA * e_latent
    out = jnp.transpose(quant.reshape(B, D1, D2, D3, C), (0, 4, 1, 2, 3))
    return (loss, out, enc)
